# Initial kernel scaffold; baseline (speedup 1.0000x reference)
import jax
import jax.numpy as jnp
from jax.experimental import pallas as pl


def kernel(x, edge_index, W1, as1, ad1, b1, W2, as2, ad2, b2, W3, as3, ad3, b3, W4, as4, ad4, b4):
    raise NotImplementedError("write your pallas kernel here")


# speedup vs baseline: 23.1424x; 23.1424x over previous
import functools

import jax
import jax.numpy as jnp
import numpy as np
from jax import lax
from jax.experimental import pallas as pl
from jax.experimental.pallas import tpu as pltpu
from jax.experimental.pallas import tpu_sc as plsc

N = 10000
NPAD = 10240
RB = 512
GRID = NPAD // RB
NW = 32
CHUNK_A = 128
CHUNK_B = 64
SROWS = NPAD // 16


def _leaky(z):
    return jnp.where(z >= 0, z, 0.2 * z)


_full = lambda s: pl.BlockSpec(s, lambda i: tuple(0 for _ in s))
_row = lambda wd: pl.BlockSpec((RB, wd), lambda i: (i, 0))
_row3 = lambda ld, wd: pl.BlockSpec((ld, RB, wd), lambda i: (0, i, 0))


def _attn_tables(h, as_ref, ad_ref, p0_ref, p1_ref, p2_ref):
    sa = jnp.dot(h, as_ref[...], preferred_element_type=jnp.float32)
    da = jnp.dot(h, ad_ref[...], preferred_element_type=jnp.float32)
    mh = _leaky(sa + da)
    return (jnp.dot(sa, p0_ref[...], preferred_element_type=jnp.float32)
            + jnp.dot(da, p1_ref[...], preferred_element_type=jnp.float32)
            + jnp.dot(mh, p2_ref[...], preferred_element_type=jnp.float32))


def _front_body(p0_ref, p1_ref, p2_ref, x_ref, w_ref, as_ref, ad_ref,
                h_ref, sm_ref):
    h = jnp.dot(x_ref[...], w_ref[...], preferred_element_type=jnp.float32)
    h_ref[...] = h
    sm_ref[...] = _attn_tables(h, as_ref, ad_ref, p0_ref, p1_ref, p2_ref)


def _tc_front(x, w, a_s, a_d, p0, p1, p2):
    outs = [
        jax.ShapeDtypeStruct((NPAD, 128), jnp.float32),
        jax.ShapeDtypeStruct((NPAD, 128), jnp.float32),
    ]
    return pl.pallas_call(
        _front_body,
        grid=(GRID,),
        in_specs=[_full(p0.shape), _full(p1.shape), _full(p2.shape),
                  _row(128), _full(w.shape), _full(a_s.shape), _full(a_d.shape)],
        out_specs=[_row(128), _row(128)],
        out_shape=outs,
    )(p0, p1, p2, x, w, a_s, a_d)


def _mid_body(residual, c40, p0_ref, p1_ref, p2_ref, r8_ref, a0_ref, a1_ref,
              s2_ref, b_ref, xp_ref, w_ref, as_ref, ad_ref, c40_ref,
              x_ref, h_ref, sm_ref):
    ssum = jnp.sum(s2_ref[...], axis=0)
    srep = jnp.dot(ssum, r8_ref[...], preferred_element_type=jnp.float32)
    u = a0_ref[...] + a1_ref[...]
    xi = jnp.maximum(u / (srep + 1e-16) + b_ref[...], 0.0)
    if residual:
        xi = xi + xp_ref[...]
    h = jnp.dot(xi, w_ref[...], preferred_element_type=jnp.float32)
    if c40:
        h = h + c40_ref[...]
    x_ref[...] = xi
    h_ref[...] = h
    sm_ref[...] = _attn_tables(h, as_ref, ad_ref, p0_ref, p1_ref, p2_ref)


def _tc_mid(a0, a1, s2, b, xprev, w, a_s, a_d, residual, c40v,
            p0, p1, p2, r8):
    outs = [
        jax.ShapeDtypeStruct((NPAD, 128), jnp.float32),
        jax.ShapeDtypeStruct((NPAD, 128), jnp.float32),
        jax.ShapeDtypeStruct((NPAD, 128), jnp.float32),
    ]
    return pl.pallas_call(
        functools.partial(_mid_body, residual, c40v is not None),
        grid=(GRID,),
        in_specs=[_full(p0.shape), _full(p1.shape), _full(p2.shape),
                  _full(r8.shape), _row(128), _row(128), _row3(2, 8),
                  _full(b.shape), _row(128),
                  _full(w.shape), _full(a_s.shape), _full(a_d.shape),
                  _full((1, 128))],
        out_specs=[_row(128), _row(128), _row(128)],
        out_shape=outs,
    )(p0, p1, p2, r8, a0, a1, s2, b, xprev, w, a_s, a_d,
      c40v if c40v is not None else jnp.zeros((1, 128), jnp.float32))


def _end_body(s40_ref, a0_ref, a1_ref, b_ref, o_ref):
    t = a0_ref[...] + a1_ref[...]
    srep = jnp.dot(t, s40_ref[...], preferred_element_type=jnp.float32)
    lg = t[:, :40] / (srep + 1e-16) + b_ref[...]
    mx = jnp.max(lg, axis=1, keepdims=True)
    ex = jnp.exp(lg - mx)
    o_ref[...] = lg - mx - jnp.log(jnp.sum(ex, axis=1, keepdims=True))


def _tc_end(a0, a1, s40, b):
    return pl.pallas_call(
        _end_body,
        grid=(GRID,),
        in_specs=[_full(s40.shape), _row(128), _row(128), _full(b.shape)],
        out_specs=_row(40),
        out_shape=jax.ShapeDtypeStruct((NPAD, 40), jnp.float32),
    )(s40, a0, a1, b)


_CP = pltpu.CompilerParams(needs_layout_passes=False)


def _mesh():
    return plsc.VectorSubcoreMesh(core_axis_name="c", subcore_axis_name="s")


def _sc_attn(src, dst, sm_t, heads, n_chunks):
    ew = n_chunks * CHUNK_B
    epad = NW * ew
    full_den = heads == 8

    def body(src_hbm, dst_hbm, sm_hbm, *rest):
        if full_den:
            (ee_hbm, sden_hbm, sh_s, src_v, dst_v, rows_s, rows_d,
             ee_buf, s_part, idx_buf, zbuf, s0, s1) = rest
        else:
            (ee_hbm, src_v, dst_v, rows_s, rows_d, ee_buf, s0, s1) = rest
        c = lax.axis_index("c")
        s = lax.axis_index("s")
        wid = s * 2 + c
        zero16 = jnp.zeros((16,), jnp.float32)
        iot = lax.iota(jnp.int32, 16)

        if full_den:
            def zsp(r, _):
                for k in range(8):
                    s_part[r, pl.ds(16 * k, 16)] = zero16
                return 0
            lax.fori_loop(0, 641, zsp, 0)

            def zrow(r, _):
                for k in range(8):
                    zbuf[r, pl.ds(16 * k, 16)] = zero16
                return 0
            lax.fori_loop(0, 8, zrow, 0)

            def zslice(i, _):
                pltpu.sync_copy(zbuf, sh_s.at[pl.ds(s * 40 + i * 8, 8)])
                return 0
            lax.fori_loop(0, 5, zslice, 0)

        def chunk(ci, _):
            base = wid * ew + ci * CHUNK_B
            pltpu.sync_copy(src_hbm.at[pl.ds(base, CHUNK_B)], src_v)
            pltpu.sync_copy(dst_hbm.at[pl.ds(base, CHUNK_B)], dst_v)
            c1 = pltpu.async_copy(sm_hbm.at[src_v], rows_s, s0)
            c2 = pltpu.async_copy(sm_hbm.at[dst_v], rows_d, s1)
            c1.wait()
            c2.wait()

            def edge(e, _):
                z = rows_s[e, pl.ds(0, 16)] + rows_d[e, pl.ds(16, 16)]
                expo = jnp.exp(_leaky(z) - rows_d[e, pl.ds(32, 16)])
                expo = jnp.where(iot < heads, expo, 0.0)
                ee_buf[pl.ds(16 * e, 16)] = expo
                if full_den:
                    d_vec = plsc.load_gather(
                        dst_v, [jnp.full((16,), e, jnp.int32)])
                    idx = d_vec * 8 + iot
                    plsc.addupdate_scatter(
                        s_part,
                        [lax.shift_right_logical(idx, 7), idx & 127],
                        expo, mask=iot < 8)
                return 0
            lax.fori_loop(0, CHUNK_B, edge, 0)
            pltpu.sync_copy(ee_buf, ee_hbm.at[pl.ds(base * 16, CHUNK_B * 16)])
            return 0
        lax.fori_loop(0, n_chunks, chunk, 0)

        if full_den:
            plsc.subcore_barrier()
            for m in range(5):
                for v in range(8):
                    idx_buf[pl.ds(16 * v, 16)] = iot + (128 * m + 16 * v)
                pltpu.sync_copy(s_part.at[pl.ds(128 * m, 128)],
                                sh_s.at[idx_buf], add=True)
            plsc.subcore_barrier()
            pltpu.sync_copy(sh_s.at[pl.ds(s * 40, 40)],
                            sden_hbm.at[c, pl.ds(s * 40, 40)])

    if full_den:
        out_type = [
            jax.ShapeDtypeStruct((epad * 16,), jnp.float32),
            jax.ShapeDtypeStruct((2, SROWS, 128), jnp.float32),
        ]
        scratch = [
            pltpu.VMEM_SHARED((SROWS, 128), jnp.float32),
            pltpu.VMEM((CHUNK_B,), jnp.int32),
            pltpu.VMEM((CHUNK_B,), jnp.int32),
            pltpu.VMEM((CHUNK_B, 128), jnp.float32),
            pltpu.VMEM((CHUNK_B, 128), jnp.float32),
            pltpu.VMEM((CHUNK_B * 16,), jnp.float32),
            pltpu.VMEM((641, 128), jnp.float32),
            pltpu.VMEM((128,), jnp.int32),
            pltpu.VMEM((8, 128), jnp.float32),
            pltpu.SemaphoreType.DMA,
            pltpu.SemaphoreType.DMA,
        ]
    else:
        out_type = [jax.ShapeDtypeStruct((epad * 16,), jnp.float32)]
        scratch = [
            pltpu.VMEM((CHUNK_B,), jnp.int32),
            pltpu.VMEM((CHUNK_B,), jnp.int32),
            pltpu.VMEM((CHUNK_B, 128), jnp.float32),
            pltpu.VMEM((CHUNK_B, 128), jnp.float32),
            pltpu.VMEM((CHUNK_B * 16,), jnp.float32),
            pltpu.SemaphoreType.DMA,
            pltpu.SemaphoreType.DMA,
        ]
    kern = pl.kernel(body, out_type=out_type, mesh=_mesh(),
                     compiler_params=_CP, scratch_types=scratch)
    return kern(src, dst, sm_t)


def _sc_agg(src, dst, ee, h_t, heads, n_chunks):
    ew = n_chunks * CHUNK_A
    rows_per_sub = NPAD // 16

    def body(src_hbm, dst_hbm, ee_hbm, h_hbm, acc_hbm,
             sh_acc, src_v, dst_v, ee_flat, rows_h, out_rows, zbuf, s0):
        c = lax.axis_index("c")
        s = lax.axis_index("s")
        wid = s * 2 + c
        zero16 = jnp.zeros((16,), jnp.float32)
        iot = lax.iota(jnp.int32, 16)

        def zrow(r, _):
            for k in range(8):
                zbuf[r, pl.ds(16 * k, 16)] = zero16
            return 0
        lax.fori_loop(0, 32, zrow, 0)

        if heads == 1:
            def zo(e, _):
                for k in range(3, 8):
                    out_rows[e, pl.ds(16 * k, 16)] = zero16
                return 0
            lax.fori_loop(0, CHUNK_A, zo, 0)

        def zslice(i, _):
            pltpu.sync_copy(
                zbuf, sh_acc.at[pl.ds(s * rows_per_sub + i * 32, 32)])
            return 0
        lax.fori_loop(0, rows_per_sub // 32, zslice, 0)
        plsc.subcore_barrier()

        def chunk(ci, _):
            base = wid * ew + ci * CHUNK_A
            pltpu.sync_copy(src_hbm.at[pl.ds(base, CHUNK_A)], src_v)
            pltpu.sync_copy(dst_hbm.at[pl.ds(base, CHUNK_A)], dst_v)
            pltpu.sync_copy(ee_hbm.at[pl.ds(base * 16, CHUNK_A * 16)],
                            ee_flat)
            c1 = pltpu.async_copy(h_hbm.at[src_v], rows_h, s0)
            c1.wait()

            def edge(e, _):
                nv = 8 if heads == 8 else 3
                for j in range(nv):
                    jj = j if heads == 8 else 0
                    bj = plsc.load_gather(
                        ee_flat, [jnp.full((16,), jj, jnp.int32) + 16 * e])
                    out_rows[e, pl.ds(16 * j, 16)] = (
                        rows_h[e, pl.ds(16 * j, 16)] * bj)
                return 0
            lax.fori_loop(0, CHUNK_A, edge, 0)
            pltpu.sync_copy(out_rows, sh_acc.at[dst_v], add=True)
            return 0
        lax.fori_loop(0, n_chunks, chunk, 0)
        plsc.subcore_barrier()
        pltpu.sync_copy(
            sh_acc.at[pl.ds(s * rows_per_sub, rows_per_sub)],
            acc_hbm.at[c, pl.ds(s * rows_per_sub, rows_per_sub)])

    kern = pl.kernel(
        body,
        out_type=jax.ShapeDtypeStruct((2, NPAD, 128), jnp.float32),
        mesh=_mesh(),
        compiler_params=_CP,
        scratch_types=[
            pltpu.VMEM_SHARED((NPAD, 128), jnp.float32),
            pltpu.VMEM((CHUNK_A,), jnp.int32),
            pltpu.VMEM((CHUNK_A,), jnp.int32),
            pltpu.VMEM((CHUNK_A * 16,), jnp.float32),
            pltpu.VMEM((CHUNK_A, 128), jnp.float32),
            pltpu.VMEM((CHUNK_A, 128), jnp.float32),
            pltpu.VMEM((32, 128), jnp.float32),
            pltpu.SemaphoreType.DMA,
        ],
    )
    return kern(src, dst, ee, h_t)


def _blockdiag(a):
    bd = (a[:, :, None] * jnp.eye(8, dtype=a.dtype)[:, None, :]).reshape(128, 8)
    return jnp.concatenate([bd, jnp.zeros((128, 8), a.dtype)], axis=1)


def _place(off):
    p = np.zeros((16, 128), np.float32)
    p[np.arange(16), off + np.arange(16)] = 1.0
    return p


_P0, _P1, _P2 = _place(0), _place(16), _place(32)
_R8 = np.kron(np.eye(8, dtype=np.float32), np.ones((1, 16), np.float32))
_S40 = np.zeros((128, 40), np.float32)
_S40[40, :] = 1.0
_C40 = np.zeros((1, 128), np.float32)
_C40[0, 40] = 1.0


def _layer_edges(src, dst, sm, h, heads, nck_a, nck_b):
    att = _sc_attn(src, dst, sm, heads, nck_b)
    if heads == 8:
        ee, sden = att
    else:
        (ee,) = att
        sden = None
    acc = _sc_agg(src, dst, ee, h, heads, nck_a)
    return acc, sden


def kernel(x, edge_index, W1, as1, ad1, b1, W2, as2, ad2, b2,
           W3, as3, ad3, b3, W4, as4, ad4, b4):
    n = x.shape[0]
    e = edge_index.shape[1]
    etot = e + n
    nck_a = -(-etot // (NW * CHUNK_A))
    nck_b = nck_a * (CHUNK_A // CHUNK_B)
    epad = NW * CHUNK_A * nck_a

    loops = jnp.arange(n, dtype=edge_index.dtype)
    pad = jnp.full((epad - etot,), n, edge_index.dtype)
    src = jnp.concatenate([edge_index[0], loops, pad])
    dst = jnp.concatenate([edge_index[1], loops, pad])

    xp = jnp.pad(x, ((0, NPAD - n), (0, 0)))
    p0, p1, p2 = jnp.asarray(_P0), jnp.asarray(_P1), jnp.asarray(_P2)
    r8 = jnp.asarray(_R8)

    as4p = jnp.zeros((128, 16), jnp.float32).at[:40, 0].set(as4[0])
    ad4p = jnp.zeros((128, 16), jnp.float32).at[:40, 0].set(ad4[0])
    w4p = jnp.pad(W4, ((0, 0), (0, 88)))

    h, sm = _tc_front(xp, W1, _blockdiag(as1), _blockdiag(ad1), p0, p1, p2)
    acc, sden = _layer_edges(src, dst, sm, h, 8, nck_a, nck_b)
    x1, h, sm = _tc_mid(acc[0], acc[1], sden.reshape(2, NPAD, 8),
                        b1.reshape(1, -1), xp, W2,
                        _blockdiag(as2), _blockdiag(ad2), False, None,
                        p0, p1, p2, r8)
    acc, sden = _layer_edges(src, dst, sm, h, 8, nck_a, nck_b)
    x2, h, sm = _tc_mid(acc[0], acc[1], sden.reshape(2, NPAD, 8),
                        b2.reshape(1, -1), x1, W3,
                        _blockdiag(as3), _blockdiag(ad3), True, None,
                        p0, p1, p2, r8)
    acc, sden = _layer_edges(src, dst, sm, h, 8, nck_a, nck_b)
    x3, h, sm = _tc_mid(acc[0], acc[1], sden.reshape(2, NPAD, 8),
                        b3.reshape(1, -1), x2, w4p, as4p, ad4p, True,
                        jnp.asarray(_C40), p0, p1, p2, r8)
    acc, _ = _layer_edges(src, dst, sm, h, 1, nck_a, nck_b)
    out = _tc_end(acc[0], acc[1], jnp.asarray(_S40), b4.reshape(1, -1))
    return out[:n]


# speedup vs baseline: 33.4042x; 1.4434x over previous
import functools

import jax
import jax.numpy as jnp
import numpy as np
from jax import lax
from jax.experimental import pallas as pl
from jax.experimental.pallas import tpu as pltpu
from jax.experimental.pallas import tpu_sc as plsc

N = 10000
NPAD = 10240
RB = 512
GRID = NPAD // RB
NW = 32
CHUNK = 64
SROWS = NPAD // 16


def _leaky(z):
    return jnp.where(z >= 0, z, 0.2 * z)


_full = lambda s: pl.BlockSpec(s, lambda i: tuple(0 for _ in s))
_row = lambda wd: pl.BlockSpec((RB, wd), lambda i: (i, 0))
_row3 = lambda ld, wd: pl.BlockSpec((ld, RB, wd), lambda i: (0, i, 0))


def _attn_tables(h, as_ref, ad_ref, p0_ref, p1_ref, p2_ref):
    sa = jnp.dot(h, as_ref[...], preferred_element_type=jnp.float32)
    da = jnp.dot(h, ad_ref[...], preferred_element_type=jnp.float32)
    mh = _leaky(sa + da)
    return (jnp.dot(sa, p0_ref[...], preferred_element_type=jnp.float32)
            + jnp.dot(da, p1_ref[...], preferred_element_type=jnp.float32)
            + jnp.dot(mh, p2_ref[...], preferred_element_type=jnp.float32))


def _front_body(p0_ref, p1_ref, p2_ref, x_ref, w_ref, as_ref, ad_ref,
                h_ref, sm_ref):
    h = jnp.dot(x_ref[...], w_ref[...], preferred_element_type=jnp.float32)
    h_ref[...] = h
    sm_ref[...] = _attn_tables(h, as_ref, ad_ref, p0_ref, p1_ref, p2_ref)


def _tc_front(x, w, a_s, a_d, p0, p1, p2):
    outs = [
        jax.ShapeDtypeStruct((NPAD, 128), jnp.float32),
        jax.ShapeDtypeStruct((NPAD, 128), jnp.float32),
    ]
    return pl.pallas_call(
        _front_body,
        grid=(GRID,),
        in_specs=[_full(p0.shape), _full(p1.shape), _full(p2.shape),
                  _row(128), _full(w.shape), _full(a_s.shape), _full(a_d.shape)],
        out_specs=[_row(128), _row(128)],
        out_shape=outs,
    )(p0, p1, p2, x, w, a_s, a_d)


def _mid_body(residual, c40, p0_ref, p1_ref, p2_ref, r8_ref, a0_ref, a1_ref,
              s2_ref, b_ref, xp_ref, w_ref, as_ref, ad_ref, c40_ref,
              x_ref, h_ref, sm_ref):
    ssum = jnp.sum(s2_ref[...], axis=0)
    srep = jnp.dot(ssum, r8_ref[...], preferred_element_type=jnp.float32)
    u = a0_ref[...] + a1_ref[...]
    xi = jnp.maximum(u / (srep + 1e-16) + b_ref[...], 0.0)
    if residual:
        xi = xi + xp_ref[...]
    h = jnp.dot(xi, w_ref[...], preferred_element_type=jnp.float32)
    if c40:
        h = h + c40_ref[...]
    x_ref[...] = xi
    h_ref[...] = h
    sm_ref[...] = _attn_tables(h, as_ref, ad_ref, p0_ref, p1_ref, p2_ref)


def _tc_mid(a0, a1, s2, b, xprev, w, a_s, a_d, residual, c40v,
            p0, p1, p2, r8):
    outs = [
        jax.ShapeDtypeStruct((NPAD, 128), jnp.float32),
        jax.ShapeDtypeStruct((NPAD, 128), jnp.float32),
        jax.ShapeDtypeStruct((NPAD, 128), jnp.float32),
    ]
    return pl.pallas_call(
        functools.partial(_mid_body, residual, c40v is not None),
        grid=(GRID,),
        in_specs=[_full(p0.shape), _full(p1.shape), _full(p2.shape),
                  _full(r8.shape), _row(128), _row(128), _row3(2, 8),
                  _full(b.shape), _row(128),
                  _full(w.shape), _full(a_s.shape), _full(a_d.shape),
                  _full((1, 128))],
        out_specs=[_row(128), _row(128), _row(128)],
        out_shape=outs,
    )(p0, p1, p2, r8, a0, a1, s2, b, xprev, w, a_s, a_d,
      c40v if c40v is not None else jnp.zeros((1, 128), jnp.float32))


def _end_body(s40_ref, a0_ref, a1_ref, b_ref, o_ref):
    t = a0_ref[...] + a1_ref[...]
    srep = jnp.dot(t, s40_ref[...], preferred_element_type=jnp.float32)
    lg = t[:, :40] / (srep + 1e-16) + b_ref[...]
    mx = jnp.max(lg, axis=1, keepdims=True)
    ex = jnp.exp(lg - mx)
    o_ref[...] = lg - mx - jnp.log(jnp.sum(ex, axis=1, keepdims=True))


def _tc_end(a0, a1, s40, b):
    return pl.pallas_call(
        _end_body,
        grid=(GRID,),
        in_specs=[_full(s40.shape), _row(128), _row(128), _full(b.shape)],
        out_specs=_row(40),
        out_shape=jax.ShapeDtypeStruct((NPAD, 40), jnp.float32),
    )(s40, a0, a1, b)


_CP = pltpu.CompilerParams(needs_layout_passes=False)


def _mesh():
    return plsc.VectorSubcoreMesh(core_axis_name="c", subcore_axis_name="s")


def _sc_attn(src, dst, sm_t, heads, n_chunks):
    ew = n_chunks * CHUNK
    epad = NW * ew
    full_den = heads == 8
    assert n_chunks % 2 == 0

    def body(src_hbm, dst_hbm, sm_hbm, *rest):
        if full_den:
            (ee_hbm, sden_hbm, sh_s, src_v0, src_v1, dst_v0, dst_v1,
             rows_s0, rows_s1, rows_d0, rows_d1, ee_buf0, ee_buf1,
             s_part, idx_buf, zbuf,
             si0, si1, sg0, sg1, sw0, sw1) = rest
        else:
            (ee_hbm, src_v0, src_v1, dst_v0, dst_v1,
             rows_s0, rows_s1, rows_d0, rows_d1, ee_buf0, ee_buf1,
             si0, si1, sg0, sg1, sw0, sw1) = rest
        src_v = (src_v0, src_v1)
        dst_v = (dst_v0, dst_v1)
        rows_s = (rows_s0, rows_s1)
        rows_d = (rows_d0, rows_d1)
        ee_buf = (ee_buf0, ee_buf1)
        si = (si0, si1)
        sg = (sg0, sg1)
        sw = (sw0, sw1)
        c = lax.axis_index("c")
        s = lax.axis_index("s")
        wid = s * 2 + c
        zero16 = jnp.zeros((16,), jnp.float32)
        iot = lax.iota(jnp.int32, 16)

        if full_den:
            def zsp(r, _):
                for k in range(8):
                    s_part[r, pl.ds(16 * k, 16)] = zero16
                return 0
            lax.fori_loop(0, 641, zsp, 0)

            def zrow(r, _):
                for k in range(8):
                    zbuf[r, pl.ds(16 * k, 16)] = zero16
                return 0
            lax.fori_loop(0, 8, zrow, 0)

            def zslice(i, _):
                pltpu.sync_copy(zbuf, sh_s.at[pl.ds(s * 40 + i * 8, 8)])
                return 0
            lax.fori_loop(0, 5, zslice, 0)

        def fire_idx(ci, p):
            base = wid * ew + ci * CHUNK
            pltpu.async_copy(src_hbm.at[pl.ds(base, CHUNK)],
                             src_v[p], si[p])
            pltpu.async_copy(dst_hbm.at[pl.ds(base, CHUNK)],
                             dst_v[p], si[p])

        def drain_idx(p):
            pltpu.make_async_copy(src_hbm.at[pl.ds(0, CHUNK)],
                                  src_v[p], si[p]).wait()
            pltpu.make_async_copy(dst_hbm.at[pl.ds(0, CHUNK)],
                                  dst_v[p], si[p]).wait()

        def fire_gather(p):
            pltpu.async_copy(sm_hbm.at[src_v[p]], rows_s[p], sg[p])
            pltpu.async_copy(sm_hbm.at[dst_v[p]], rows_d[p], sg[p])

        def drain_gather(p):
            pltpu.make_async_copy(sm_hbm.at[pl.ds(0, CHUNK)],
                                  rows_s[p], sg[p]).wait()
            pltpu.make_async_copy(sm_hbm.at[pl.ds(0, CHUNK)],
                                  rows_d[p], sg[p]).wait()

        def drain_write(p):
            pltpu.make_async_copy(ee_hbm.at[pl.ds(0, CHUNK * 16)],
                                  ee_buf[p], sw[p]).wait()

        def compute(p, ci):
            def edge(e, _):
                z = rows_s[p][e, pl.ds(0, 16)] + rows_d[p][e, pl.ds(16, 16)]
                expo = jnp.exp(_leaky(z) - rows_d[p][e, pl.ds(32, 16)])
                expo = jnp.where(iot < heads, expo, 0.0)
                ee_buf[p][pl.ds(16 * e, 16)] = expo
                if full_den:
                    d_vec = plsc.load_gather(
                        dst_v[p], [jnp.full((16,), e, jnp.int32)])
                    idx = d_vec * 8 + iot
                    plsc.addupdate_scatter(
                        s_part,
                        [lax.shift_right_logical(idx, 7), idx & 127],
                        expo, mask=iot < 8)
                return 0
            lax.fori_loop(0, CHUNK, edge, 0)
            base = wid * ew + ci * CHUNK
            pltpu.async_copy(ee_buf[p],
                             ee_hbm.at[pl.ds(base * 16, CHUNK * 16)], sw[p])

        fire_idx(0, 0)
        drain_idx(0)
        fire_gather(0)
        fire_idx(1, 1)

        def step(i, _):
            ci = 2 * i
            drain_idx(1)
            fire_gather(1)
            drain_gather(0)

            @pl.when(ci >= 2)
            def _():
                drain_write(0)
            compute(0, ci)

            @pl.when(ci + 2 < n_chunks)
            def _():
                fire_idx(ci + 2, 0)

            @pl.when(ci + 2 < n_chunks)
            def _():
                drain_idx(0)
                fire_gather(0)
            drain_gather(1)

            @pl.when(ci >= 2)
            def _():
                drain_write(1)
            compute(1, ci + 1)

            @pl.when(ci + 3 < n_chunks)
            def _():
                fire_idx(ci + 3, 1)
            return 0
        lax.fori_loop(0, n_chunks // 2, step, 0)
        drain_write(0)
        drain_write(1)

        if full_den:
            plsc.subcore_barrier()
            for m in range(5):
                for v in range(8):
                    idx_buf[pl.ds(16 * v, 16)] = iot + (128 * m + 16 * v)
                pltpu.sync_copy(s_part.at[pl.ds(128 * m, 128)],
                                sh_s.at[idx_buf], add=True)
            plsc.subcore_barrier()
            pltpu.sync_copy(sh_s.at[pl.ds(s * 40, 40)],
                            sden_hbm.at[c, pl.ds(s * 40, 40)])

    sems = [pltpu.SemaphoreType.DMA] * 6
    if full_den:
        out_type = [
            jax.ShapeDtypeStruct((epad * 16,), jnp.float32),
            jax.ShapeDtypeStruct((2, SROWS, 128), jnp.float32),
        ]
        scratch = [
            pltpu.VMEM_SHARED((SROWS, 128), jnp.float32),
            pltpu.VMEM((CHUNK,), jnp.int32),
            pltpu.VMEM((CHUNK,), jnp.int32),
            pltpu.VMEM((CHUNK,), jnp.int32),
            pltpu.VMEM((CHUNK,), jnp.int32),
            pltpu.VMEM((CHUNK, 128), jnp.float32),
            pltpu.VMEM((CHUNK, 128), jnp.float32),
            pltpu.VMEM((CHUNK, 128), jnp.float32),
            pltpu.VMEM((CHUNK, 128), jnp.float32),
            pltpu.VMEM((CHUNK * 16,), jnp.float32),
            pltpu.VMEM((CHUNK * 16,), jnp.float32),
            pltpu.VMEM((641, 128), jnp.float32),
            pltpu.VMEM((128,), jnp.int32),
            pltpu.VMEM((8, 128), jnp.float32),
        ] + sems
    else:
        out_type = [jax.ShapeDtypeStruct((epad * 16,), jnp.float32)]
        scratch = [
            pltpu.VMEM((CHUNK,), jnp.int32),
            pltpu.VMEM((CHUNK,), jnp.int32),
            pltpu.VMEM((CHUNK,), jnp.int32),
            pltpu.VMEM((CHUNK,), jnp.int32),
            pltpu.VMEM((CHUNK, 128), jnp.float32),
            pltpu.VMEM((CHUNK, 128), jnp.float32),
            pltpu.VMEM((CHUNK, 128), jnp.float32),
            pltpu.VMEM((CHUNK, 128), jnp.float32),
            pltpu.VMEM((CHUNK * 16,), jnp.float32),
            pltpu.VMEM((CHUNK * 16,), jnp.float32),
        ] + sems
    kern = pl.kernel(body, out_type=out_type, mesh=_mesh(),
                     compiler_params=_CP, scratch_types=scratch)
    return kern(src, dst, sm_t)


def _sc_agg(src, dst, ee, h_t, heads, n_chunks):
    ew = n_chunks * CHUNK
    rows_per_sub = NPAD // 16
    assert n_chunks % 2 == 0

    def body(src_hbm, dst_hbm, ee_hbm, h_hbm, acc_hbm,
             sh_acc, src_v0, src_v1, dst_v0, dst_v1, ee_flat0, ee_flat1,
             rows_h0, rows_h1, out_rows0, out_rows1, zbuf,
             si0, si1, sg0, sg1):
        src_v = (src_v0, src_v1)
        dst_v = (dst_v0, dst_v1)
        ee_flat = (ee_flat0, ee_flat1)
        rows_h = (rows_h0, rows_h1)
        out_rows = (out_rows0, out_rows1)
        si = (si0, si1)
        sg = (sg0, sg1)
        c = lax.axis_index("c")
        s = lax.axis_index("s")
        wid = s * 2 + c
        zero16 = jnp.zeros((16,), jnp.float32)

        def zrow(r, _):
            for k in range(8):
                zbuf[r, pl.ds(16 * k, 16)] = zero16
            return 0
        lax.fori_loop(0, 8, zrow, 0)

        if heads == 1:
            def zo(e, _):
                for p in range(2):
                    for k in range(3, 8):
                        out_rows[p][e, pl.ds(16 * k, 16)] = zero16
                return 0
            lax.fori_loop(0, CHUNK, zo, 0)

        def zslice(i, _):
            pltpu.sync_copy(
                zbuf, sh_acc.at[pl.ds(s * rows_per_sub + i * 8, 8)])
            return 0
        lax.fori_loop(0, rows_per_sub // 8, zslice, 0)
        plsc.subcore_barrier()

        def fire_idx(ci, p):
            base = wid * ew + ci * CHUNK
            pltpu.async_copy(src_hbm.at[pl.ds(base, CHUNK)],
                             src_v[p], si[p])
            pltpu.async_copy(dst_hbm.at[pl.ds(base, CHUNK)],
                             dst_v[p], si[p])
            pltpu.async_copy(ee_hbm.at[pl.ds(base * 16, CHUNK * 16)],
                             ee_flat[p], si[p])

        def drain_idx(p):
            pltpu.make_async_copy(src_hbm.at[pl.ds(0, CHUNK)],
                                  src_v[p], si[p]).wait()
            pltpu.make_async_copy(dst_hbm.at[pl.ds(0, CHUNK)],
                                  dst_v[p], si[p]).wait()
            pltpu.make_async_copy(ee_hbm.at[pl.ds(0, CHUNK * 16)],
                                  ee_flat[p], si[p]).wait()

        def fire_gather(p):
            pltpu.async_copy(h_hbm.at[src_v[p]], rows_h[p], sg[p])

        def drain_gather(p):
            pltpu.make_async_copy(h_hbm.at[pl.ds(0, CHUNK)],
                                  rows_h[p], sg[p]).wait()

        def compute(p, ci):
            def edge(e, _):
                nv = 8 if heads == 8 else 3
                for j in range(nv):
                    jj = j if heads == 8 else 0
                    bj = plsc.load_gather(
                        ee_flat[p],
                        [jnp.full((16,), jj, jnp.int32) + 16 * e])
                    out_rows[p][e, pl.ds(16 * j, 16)] = (
                        rows_h[p][e, pl.ds(16 * j, 16)] * bj)
                return 0
            lax.fori_loop(0, CHUNK, edge, 0)
            pltpu.sync_copy(out_rows[p], sh_acc.at[dst_v[p]], add=True)

        fire_idx(0, 0)
        drain_idx(0)
        fire_gather(0)
        fire_idx(1, 1)

        def step(i, _):
            ci = 2 * i
            drain_idx(1)
            fire_gather(1)
            drain_gather(0)
            compute(0, ci)

            @pl.when(ci + 2 < n_chunks)
            def _():
                fire_idx(ci + 2, 0)

            @pl.when(ci + 2 < n_chunks)
            def _():
                drain_idx(0)
                fire_gather(0)
            drain_gather(1)
            compute(1, ci + 1)

            @pl.when(ci + 3 < n_chunks)
            def _():
                fire_idx(ci + 3, 1)
            return 0
        lax.fori_loop(0, n_chunks // 2, step, 0)
        plsc.subcore_barrier()
        pltpu.sync_copy(
            sh_acc.at[pl.ds(s * rows_per_sub, rows_per_sub)],
            acc_hbm.at[c, pl.ds(s * rows_per_sub, rows_per_sub)])

    kern = pl.kernel(
        body,
        out_type=jax.ShapeDtypeStruct((2, NPAD, 128), jnp.float32),
        mesh=_mesh(),
        compiler_params=_CP,
        scratch_types=[
            pltpu.VMEM_SHARED((NPAD, 128), jnp.float32),
            pltpu.VMEM((CHUNK,), jnp.int32),
            pltpu.VMEM((CHUNK,), jnp.int32),
            pltpu.VMEM((CHUNK,), jnp.int32),
            pltpu.VMEM((CHUNK,), jnp.int32),
            pltpu.VMEM((CHUNK * 16,), jnp.float32),
            pltpu.VMEM((CHUNK * 16,), jnp.float32),
            pltpu.VMEM((CHUNK, 128), jnp.float32),
            pltpu.VMEM((CHUNK, 128), jnp.float32),
            pltpu.VMEM((CHUNK, 128), jnp.float32),
            pltpu.VMEM((CHUNK, 128), jnp.float32),
            pltpu.VMEM((8, 128), jnp.float32),
            pltpu.SemaphoreType.DMA,
            pltpu.SemaphoreType.DMA,
            pltpu.SemaphoreType.DMA,
            pltpu.SemaphoreType.DMA,
        ],
    )
    return kern(src, dst, ee, h_t)


def _blockdiag(a):
    bd = (a[:, :, None] * jnp.eye(8, dtype=a.dtype)[:, None, :]).reshape(128, 8)
    return jnp.concatenate([bd, jnp.zeros((128, 8), a.dtype)], axis=1)


def _place(off):
    p = np.zeros((16, 128), np.float32)
    p[np.arange(16), off + np.arange(16)] = 1.0
    return p


_P0, _P1, _P2 = _place(0), _place(16), _place(32)
_R8 = np.kron(np.eye(8, dtype=np.float32), np.ones((1, 16), np.float32))
_S40 = np.zeros((128, 40), np.float32)
_S40[40, :] = 1.0
_C40 = np.zeros((1, 128), np.float32)
_C40[0, 40] = 1.0


def _layer_edges(src, dst, sm, h, heads, nck):
    att = _sc_attn(src, dst, sm, heads, nck)
    if heads == 8:
        ee, sden = att
    else:
        (ee,) = att
        sden = None
    acc = _sc_agg(src, dst, ee, h, heads, nck)
    return acc, sden


def kernel(x, edge_index, W1, as1, ad1, b1, W2, as2, ad2, b2,
           W3, as3, ad3, b3, W4, as4, ad4, b4):
    n = x.shape[0]
    e = edge_index.shape[1]
    etot = e + n
    nck = -(-etot // (NW * CHUNK))
    nck += nck % 2
    epad = NW * CHUNK * nck

    loops = jnp.arange(n, dtype=edge_index.dtype)
    pad = jnp.full((epad - etot,), n, edge_index.dtype)
    src = jnp.concatenate([edge_index[0], loops, pad])
    dst = jnp.concatenate([edge_index[1], loops, pad])

    xp = jnp.pad(x, ((0, NPAD - n), (0, 0)))
    p0, p1, p2 = jnp.asarray(_P0), jnp.asarray(_P1), jnp.asarray(_P2)
    r8 = jnp.asarray(_R8)

    as4p = jnp.zeros((128, 16), jnp.float32).at[:40, 0].set(as4[0])
    ad4p = jnp.zeros((128, 16), jnp.float32).at[:40, 0].set(ad4[0])
    w4p = jnp.pad(W4, ((0, 0), (0, 88)))

    h, sm = _tc_front(xp, W1, _blockdiag(as1), _blockdiag(ad1), p0, p1, p2)
    acc, sden = _layer_edges(src, dst, sm, h, 8, nck)
    x1, h, sm = _tc_mid(acc[0], acc[1], sden.reshape(2, NPAD, 8),
                        b1.reshape(1, -1), xp, W2,
                        _blockdiag(as2), _blockdiag(ad2), False, None,
                        p0, p1, p2, r8)
    acc, sden = _layer_edges(src, dst, sm, h, 8, nck)
    x2, h, sm = _tc_mid(acc[0], acc[1], sden.reshape(2, NPAD, 8),
                        b2.reshape(1, -1), x1, W3,
                        _blockdiag(as3), _blockdiag(ad3), True, None,
                        p0, p1, p2, r8)
    acc, sden = _layer_edges(src, dst, sm, h, 8, nck)
    x3, h, sm = _tc_mid(acc[0], acc[1], sden.reshape(2, NPAD, 8),
                        b3.reshape(1, -1), x2, w4p, as4p, ad4p, True,
                        jnp.asarray(_C40), p0, p1, p2, r8)
    acc, _ = _layer_edges(src, dst, sm, h, 1, nck)
    out = _tc_end(acc[0], acc[1], jnp.asarray(_S40), b4.reshape(1, -1))
    return out[:n]


# speedup vs baseline: 48.6546x; 1.4565x over previous
import functools

import jax
import jax.numpy as jnp
import numpy as np
from jax import lax
from jax.experimental import pallas as pl
from jax.experimental.pallas import tpu as pltpu
from jax.experimental.pallas import tpu_sc as plsc

N = 10000
NPAD = 10240
RB = 512
GRID = NPAD // RB
NW = 32
CHUNK = 64
SROWS = NPAD // 16


def _leaky(z):
    return jnp.where(z >= 0, z, 0.2 * z)


_full = lambda s: pl.BlockSpec(s, lambda i: tuple(0 for _ in s))
_row = lambda wd: pl.BlockSpec((RB, wd), lambda i: (i, 0))
_row3 = lambda ld, wd: pl.BlockSpec((ld, RB, wd), lambda i: (0, i, 0))


def _attn_tables(h, as_ref, ad_ref, p0_ref, p1_ref, p2_ref):
    sa = jnp.dot(h, as_ref[...], preferred_element_type=jnp.float32)
    da = jnp.dot(h, ad_ref[...], preferred_element_type=jnp.float32)
    mh = _leaky(sa + da)
    return (jnp.dot(sa, p0_ref[...], preferred_element_type=jnp.float32)
            + jnp.dot(da, p1_ref[...], preferred_element_type=jnp.float32)
            + jnp.dot(mh, p2_ref[...], preferred_element_type=jnp.float32))


def _front_body(p0_ref, p1_ref, p2_ref, x_ref, w_ref, as_ref, ad_ref,
                h_ref, sm_ref):
    h = jnp.dot(x_ref[...], w_ref[...], preferred_element_type=jnp.float32)
    h_ref[...] = h
    sm_ref[...] = _attn_tables(h, as_ref, ad_ref, p0_ref, p1_ref, p2_ref)


def _tc_front(x, w, a_s, a_d, p0, p1, p2):
    outs = [
        jax.ShapeDtypeStruct((NPAD, 128), jnp.float32),
        jax.ShapeDtypeStruct((NPAD, 128), jnp.float32),
    ]
    return pl.pallas_call(
        _front_body,
        grid=(GRID,),
        in_specs=[_full(p0.shape), _full(p1.shape), _full(p2.shape),
                  _row(128), _full(w.shape), _full(a_s.shape), _full(a_d.shape)],
        out_specs=[_row(128), _row(128)],
        out_shape=outs,
    )(p0, p1, p2, x, w, a_s, a_d)


def _mid_body(residual, c40, p0_ref, p1_ref, p2_ref, r8_ref, a0_ref, a1_ref,
              s2_ref, b_ref, xp_ref, w_ref, as_ref, ad_ref, c40_ref,
              x_ref, h_ref, sm_ref):
    ssum = jnp.sum(s2_ref[...], axis=0)
    srep = jnp.dot(ssum, r8_ref[...], preferred_element_type=jnp.float32)
    u = a0_ref[...] + a1_ref[...]
    xi = jnp.maximum(u / (srep + 1e-16) + b_ref[...], 0.0)
    if residual:
        xi = xi + xp_ref[...]
    h = jnp.dot(xi, w_ref[...], preferred_element_type=jnp.float32)
    if c40:
        h = h + c40_ref[...]
    x_ref[...] = xi
    h_ref[...] = h
    sm_ref[...] = _attn_tables(h, as_ref, ad_ref, p0_ref, p1_ref, p2_ref)


def _tc_mid(a0, a1, s2, b, xprev, w, a_s, a_d, residual, c40v,
            p0, p1, p2, r8):
    outs = [
        jax.ShapeDtypeStruct((NPAD, 128), jnp.float32),
        jax.ShapeDtypeStruct((NPAD, 128), jnp.float32),
        jax.ShapeDtypeStruct((NPAD, 128), jnp.float32),
    ]
    return pl.pallas_call(
        functools.partial(_mid_body, residual, c40v is not None),
        grid=(GRID,),
        in_specs=[_full(p0.shape), _full(p1.shape), _full(p2.shape),
                  _full(r8.shape), _row(128), _row(128), _row3(2, 8),
                  _full(b.shape), _row(128),
                  _full(w.shape), _full(a_s.shape), _full(a_d.shape),
                  _full((1, 128))],
        out_specs=[_row(128), _row(128), _row(128)],
        out_shape=outs,
    )(p0, p1, p2, r8, a0, a1, s2, b, xprev, w, a_s, a_d,
      c40v if c40v is not None else jnp.zeros((1, 128), jnp.float32))


def _end_body(s40_ref, a0_ref, a1_ref, b_ref, o_ref):
    t = a0_ref[...] + a1_ref[...]
    srep = jnp.dot(t, s40_ref[...], preferred_element_type=jnp.float32)
    lg = t[:, :40] / (srep + 1e-16) + b_ref[...]
    mx = jnp.max(lg, axis=1, keepdims=True)
    ex = jnp.exp(lg - mx)
    o_ref[...] = lg - mx - jnp.log(jnp.sum(ex, axis=1, keepdims=True))


def _tc_end(a0, a1, s40, b):
    return pl.pallas_call(
        _end_body,
        grid=(GRID,),
        in_specs=[_full(s40.shape), _row(128), _row(128), _full(b.shape)],
        out_specs=_row(40),
        out_shape=jax.ShapeDtypeStruct((NPAD, 40), jnp.float32),
    )(s40, a0, a1, b)


_CP = pltpu.CompilerParams(needs_layout_passes=False)


def _mesh():
    return plsc.VectorSubcoreMesh(core_axis_name="c", subcore_axis_name="s")


def _sc_attn(src, dst, sm_t, heads, n_chunks):
    ew = n_chunks * CHUNK
    epad = NW * ew
    full_den = heads == 8
    assert n_chunks % 2 == 0

    def body(src_hbm, dst_hbm, sm_hbm, *rest):
        if full_den:
            (ee_hbm, sden_hbm, sh_s, src_v0, src_v1, dst_v0, dst_v1,
             rows_s0, rows_s1, rows_d0, rows_d1, ee_buf0, ee_buf1,
             s_part, idx_buf, zbuf,
             si0, si1, sg0, sg1, sw0, sw1) = rest
        else:
            (ee_hbm, src_v0, src_v1, dst_v0, dst_v1,
             rows_s0, rows_s1, rows_d0, rows_d1, ee_buf0, ee_buf1,
             si0, si1, sg0, sg1, sw0, sw1) = rest
        src_v = (src_v0, src_v1)
        dst_v = (dst_v0, dst_v1)
        rows_s = (rows_s0, rows_s1)
        rows_d = (rows_d0, rows_d1)
        ee_buf = (ee_buf0, ee_buf1)
        si = (si0, si1)
        sg = (sg0, sg1)
        sw = (sw0, sw1)
        c = lax.axis_index("c")
        s = lax.axis_index("s")
        wid = s * 2 + c
        zero16 = jnp.zeros((16,), jnp.float32)
        iot = lax.iota(jnp.int32, 16)

        if full_den:
            def zsp(r, _):
                for k in range(8):
                    s_part[r, pl.ds(16 * k, 16)] = zero16
                return 0
            lax.fori_loop(0, 641, zsp, 0)

            def zrow(r, _):
                for k in range(8):
                    zbuf[r, pl.ds(16 * k, 16)] = zero16
                return 0
            lax.fori_loop(0, 8, zrow, 0)

            def zslice(i, _):
                pltpu.sync_copy(zbuf, sh_s.at[pl.ds(s * 40 + i * 8, 8)])
                return 0
            lax.fori_loop(0, 5, zslice, 0)

        def fire_idx(ci, p):
            base = wid * ew + ci * CHUNK
            pltpu.async_copy(src_hbm.at[pl.ds(base, CHUNK)],
                             src_v[p], si[p])
            pltpu.async_copy(dst_hbm.at[pl.ds(base, CHUNK)],
                             dst_v[p], si[p])

        def drain_idx(p):
            pltpu.make_async_copy(src_hbm.at[pl.ds(0, CHUNK)],
                                  src_v[p], si[p]).wait()
            pltpu.make_async_copy(dst_hbm.at[pl.ds(0, CHUNK)],
                                  dst_v[p], si[p]).wait()

        def fire_gather(p):
            pltpu.async_copy(sm_hbm.at[src_v[p]], rows_s[p], sg[p])
            pltpu.async_copy(sm_hbm.at[dst_v[p]], rows_d[p], sg[p])

        def drain_gather(p):
            pltpu.make_async_copy(sm_hbm.at[pl.ds(0, CHUNK)],
                                  rows_s[p], sg[p]).wait()
            pltpu.make_async_copy(sm_hbm.at[pl.ds(0, CHUNK)],
                                  rows_d[p], sg[p]).wait()

        def drain_write(p):
            pltpu.make_async_copy(ee_hbm.at[pl.ds(0, CHUNK * 16)],
                                  ee_buf[p], sw[p]).wait()

        def compute(p, ci):
            @plsc.parallel_loop(0, CHUNK, unroll=2)
            def edge(e):
                z = rows_s[p][e, pl.ds(0, 16)] + rows_d[p][e, pl.ds(16, 16)]
                expo = jnp.exp(_leaky(z) - rows_d[p][e, pl.ds(32, 16)])
                expo = jnp.where(iot < heads, expo, 0.0)
                ee_buf[p][pl.ds(16 * e, 16)] = expo
                if full_den:
                    d_vec = plsc.load_gather(
                        dst_v[p], [jnp.full((16,), e, jnp.int32)])
                    idx = d_vec * 8 + iot
                    plsc.addupdate_scatter(
                        s_part,
                        [lax.shift_right_logical(idx, 7), idx & 127],
                        expo, mask=iot < 8)
            base = wid * ew + ci * CHUNK
            pltpu.async_copy(ee_buf[p],
                             ee_hbm.at[pl.ds(base * 16, CHUNK * 16)], sw[p])

        fire_idx(0, 0)
        drain_idx(0)
        fire_gather(0)
        fire_idx(1, 1)

        def step(i, _):
            ci = 2 * i
            drain_idx(1)
            fire_gather(1)
            drain_gather(0)

            @pl.when(ci >= 2)
            def _():
                drain_write(0)
            compute(0, ci)

            @pl.when(ci + 2 < n_chunks)
            def _():
                fire_idx(ci + 2, 0)

            @pl.when(ci + 2 < n_chunks)
            def _():
                drain_idx(0)
                fire_gather(0)
            drain_gather(1)

            @pl.when(ci >= 2)
            def _():
                drain_write(1)
            compute(1, ci + 1)

            @pl.when(ci + 3 < n_chunks)
            def _():
                fire_idx(ci + 3, 1)
            return 0
        lax.fori_loop(0, n_chunks // 2, step, 0)
        drain_write(0)
        drain_write(1)

        if full_den:
            plsc.subcore_barrier()
            for m in range(5):
                for v in range(8):
                    idx_buf[pl.ds(16 * v, 16)] = iot + (128 * m + 16 * v)
                pltpu.sync_copy(s_part.at[pl.ds(128 * m, 128)],
                                sh_s.at[idx_buf], add=True)
            plsc.subcore_barrier()
            pltpu.sync_copy(sh_s.at[pl.ds(s * 40, 40)],
                            sden_hbm.at[c, pl.ds(s * 40, 40)])

    sems = [pltpu.SemaphoreType.DMA] * 6
    if full_den:
        out_type = [
            jax.ShapeDtypeStruct((epad * 16,), jnp.float32),
            jax.ShapeDtypeStruct((2, SROWS, 128), jnp.float32),
        ]
        scratch = [
            pltpu.VMEM_SHARED((SROWS, 128), jnp.float32),
            pltpu.VMEM((CHUNK,), jnp.int32),
            pltpu.VMEM((CHUNK,), jnp.int32),
            pltpu.VMEM((CHUNK,), jnp.int32),
            pltpu.VMEM((CHUNK,), jnp.int32),
            pltpu.VMEM((CHUNK, 128), jnp.float32),
            pltpu.VMEM((CHUNK, 128), jnp.float32),
            pltpu.VMEM((CHUNK, 128), jnp.float32),
            pltpu.VMEM((CHUNK, 128), jnp.float32),
            pltpu.VMEM((CHUNK * 16,), jnp.float32),
            pltpu.VMEM((CHUNK * 16,), jnp.float32),
            pltpu.VMEM((641, 128), jnp.float32),
            pltpu.VMEM((128,), jnp.int32),
            pltpu.VMEM((8, 128), jnp.float32),
        ] + sems
    else:
        out_type = [jax.ShapeDtypeStruct((epad * 16,), jnp.float32)]
        scratch = [
            pltpu.VMEM((CHUNK,), jnp.int32),
            pltpu.VMEM((CHUNK,), jnp.int32),
            pltpu.VMEM((CHUNK,), jnp.int32),
            pltpu.VMEM((CHUNK,), jnp.int32),
            pltpu.VMEM((CHUNK, 128), jnp.float32),
            pltpu.VMEM((CHUNK, 128), jnp.float32),
            pltpu.VMEM((CHUNK, 128), jnp.float32),
            pltpu.VMEM((CHUNK, 128), jnp.float32),
            pltpu.VMEM((CHUNK * 16,), jnp.float32),
            pltpu.VMEM((CHUNK * 16,), jnp.float32),
        ] + sems
    kern = pl.kernel(body, out_type=out_type, mesh=_mesh(),
                     compiler_params=_CP, scratch_types=scratch)
    return kern(src, dst, sm_t)


def _sc_agg(src, dst, ee, h_t, heads, n_chunks):
    ew = n_chunks * CHUNK
    rows_per_sub = NPAD // 16
    assert n_chunks % 2 == 0

    def body(src_hbm, dst_hbm, ee_hbm, h_hbm, acc_hbm,
             sh_acc, src_v0, src_v1, dst_v0, dst_v1, ee_flat0, ee_flat1,
             rows_h0, rows_h1, out_rows0, out_rows1, zbuf,
             si0, si1, sg0, sg1):
        src_v = (src_v0, src_v1)
        dst_v = (dst_v0, dst_v1)
        ee_flat = (ee_flat0, ee_flat1)
        rows_h = (rows_h0, rows_h1)
        out_rows = (out_rows0, out_rows1)
        si = (si0, si1)
        sg = (sg0, sg1)
        c = lax.axis_index("c")
        s = lax.axis_index("s")
        wid = s * 2 + c
        zero16 = jnp.zeros((16,), jnp.float32)

        def zrow(r, _):
            for k in range(8):
                zbuf[r, pl.ds(16 * k, 16)] = zero16
            return 0
        lax.fori_loop(0, 8, zrow, 0)

        if heads == 1:
            def zo(e, _):
                for p in range(2):
                    for k in range(3, 8):
                        out_rows[p][e, pl.ds(16 * k, 16)] = zero16
                return 0
            lax.fori_loop(0, CHUNK, zo, 0)

        def zslice(i, _):
            pltpu.sync_copy(
                zbuf, sh_acc.at[pl.ds(s * rows_per_sub + i * 8, 8)])
            return 0
        lax.fori_loop(0, rows_per_sub // 8, zslice, 0)
        plsc.subcore_barrier()

        def fire_idx(ci, p):
            base = wid * ew + ci * CHUNK
            pltpu.async_copy(src_hbm.at[pl.ds(base, CHUNK)],
                             src_v[p], si[p])
            pltpu.async_copy(dst_hbm.at[pl.ds(base, CHUNK)],
                             dst_v[p], si[p])
            pltpu.async_copy(ee_hbm.at[pl.ds(base * 16, CHUNK * 16)],
                             ee_flat[p], si[p])

        def drain_idx(p):
            pltpu.make_async_copy(src_hbm.at[pl.ds(0, CHUNK)],
                                  src_v[p], si[p]).wait()
            pltpu.make_async_copy(dst_hbm.at[pl.ds(0, CHUNK)],
                                  dst_v[p], si[p]).wait()
            pltpu.make_async_copy(ee_hbm.at[pl.ds(0, CHUNK * 16)],
                                  ee_flat[p], si[p]).wait()

        def fire_gather(p):
            pltpu.async_copy(h_hbm.at[src_v[p]], rows_h[p], sg[p])

        def drain_gather(p):
            pltpu.make_async_copy(h_hbm.at[pl.ds(0, CHUNK)],
                                  rows_h[p], sg[p]).wait()

        def compute(p, ci):
            @plsc.parallel_loop(0, CHUNK, unroll=2)
            def edge(e):
                nv = 8 if heads == 8 else 3
                for j in range(nv):
                    jj = j if heads == 8 else 0
                    bj = plsc.load_gather(
                        ee_flat[p],
                        [jnp.full((16,), jj, jnp.int32) + 16 * e])
                    out_rows[p][e, pl.ds(16 * j, 16)] = (
                        rows_h[p][e, pl.ds(16 * j, 16)] * bj)
            pltpu.sync_copy(out_rows[p], sh_acc.at[dst_v[p]], add=True)

        fire_idx(0, 0)
        drain_idx(0)
        fire_gather(0)
        fire_idx(1, 1)

        def step(i, _):
            ci = 2 * i
            drain_idx(1)
            fire_gather(1)
            drain_gather(0)
            compute(0, ci)

            @pl.when(ci + 2 < n_chunks)
            def _():
                fire_idx(ci + 2, 0)

            @pl.when(ci + 2 < n_chunks)
            def _():
                drain_idx(0)
                fire_gather(0)
            drain_gather(1)
            compute(1, ci + 1)

            @pl.when(ci + 3 < n_chunks)
            def _():
                fire_idx(ci + 3, 1)
            return 0
        lax.fori_loop(0, n_chunks // 2, step, 0)
        plsc.subcore_barrier()
        pltpu.sync_copy(
            sh_acc.at[pl.ds(s * rows_per_sub, rows_per_sub)],
            acc_hbm.at[c, pl.ds(s * rows_per_sub, rows_per_sub)])

    kern = pl.kernel(
        body,
        out_type=jax.ShapeDtypeStruct((2, NPAD, 128), jnp.float32),
        mesh=_mesh(),
        compiler_params=_CP,
        scratch_types=[
            pltpu.VMEM_SHARED((NPAD, 128), jnp.float32),
            pltpu.VMEM((CHUNK,), jnp.int32),
            pltpu.VMEM((CHUNK,), jnp.int32),
            pltpu.VMEM((CHUNK,), jnp.int32),
            pltpu.VMEM((CHUNK,), jnp.int32),
            pltpu.VMEM((CHUNK * 16,), jnp.float32),
            pltpu.VMEM((CHUNK * 16,), jnp.float32),
            pltpu.VMEM((CHUNK, 128), jnp.float32),
            pltpu.VMEM((CHUNK, 128), jnp.float32),
            pltpu.VMEM((CHUNK, 128), jnp.float32),
            pltpu.VMEM((CHUNK, 128), jnp.float32),
            pltpu.VMEM((8, 128), jnp.float32),
            pltpu.SemaphoreType.DMA,
            pltpu.SemaphoreType.DMA,
            pltpu.SemaphoreType.DMA,
            pltpu.SemaphoreType.DMA,
        ],
    )
    return kern(src, dst, ee, h_t)


def _blockdiag(a):
    bd = (a[:, :, None] * jnp.eye(8, dtype=a.dtype)[:, None, :]).reshape(128, 8)
    return jnp.concatenate([bd, jnp.zeros((128, 8), a.dtype)], axis=1)


def _place(off):
    p = np.zeros((16, 128), np.float32)
    p[np.arange(16), off + np.arange(16)] = 1.0
    return p


_P0, _P1, _P2 = _place(0), _place(16), _place(32)
_R8 = np.kron(np.eye(8, dtype=np.float32), np.ones((1, 16), np.float32))
_S40 = np.zeros((128, 40), np.float32)
_S40[40, :] = 1.0
_C40 = np.zeros((1, 128), np.float32)
_C40[0, 40] = 1.0


def _layer_edges(src, dst, sm, h, heads, nck):
    att = _sc_attn(src, dst, sm, heads, nck)
    if heads == 8:
        ee, sden = att
    else:
        (ee,) = att
        sden = None
    acc = _sc_agg(src, dst, ee, h, heads, nck)
    return acc, sden


def kernel(x, edge_index, W1, as1, ad1, b1, W2, as2, ad2, b2,
           W3, as3, ad3, b3, W4, as4, ad4, b4):
    n = x.shape[0]
    e = edge_index.shape[1]
    etot = e + n
    nck = -(-etot // (NW * CHUNK))
    nck += nck % 2
    epad = NW * CHUNK * nck

    loops = jnp.arange(n, dtype=edge_index.dtype)
    pad = jnp.full((epad - etot,), n, edge_index.dtype)
    src = jnp.concatenate([edge_index[0], loops, pad])
    dst = jnp.concatenate([edge_index[1], loops, pad])

    xp = jnp.pad(x, ((0, NPAD - n), (0, 0)))
    p0, p1, p2 = jnp.asarray(_P0), jnp.asarray(_P1), jnp.asarray(_P2)
    r8 = jnp.asarray(_R8)

    as4p = jnp.zeros((128, 16), jnp.float32).at[:40, 0].set(as4[0])
    ad4p = jnp.zeros((128, 16), jnp.float32).at[:40, 0].set(ad4[0])
    w4p = jnp.pad(W4, ((0, 0), (0, 88)))

    h, sm = _tc_front(xp, W1, _blockdiag(as1), _blockdiag(ad1), p0, p1, p2)
    acc, sden = _layer_edges(src, dst, sm, h, 8, nck)
    x1, h, sm = _tc_mid(acc[0], acc[1], sden.reshape(2, NPAD, 8),
                        b1.reshape(1, -1), xp, W2,
                        _blockdiag(as2), _blockdiag(ad2), False, None,
                        p0, p1, p2, r8)
    acc, sden = _layer_edges(src, dst, sm, h, 8, nck)
    x2, h, sm = _tc_mid(acc[0], acc[1], sden.reshape(2, NPAD, 8),
                        b2.reshape(1, -1), x1, W3,
                        _blockdiag(as3), _blockdiag(ad3), True, None,
                        p0, p1, p2, r8)
    acc, sden = _layer_edges(src, dst, sm, h, 8, nck)
    x3, h, sm = _tc_mid(acc[0], acc[1], sden.reshape(2, NPAD, 8),
                        b3.reshape(1, -1), x2, w4p, as4p, ad4p, True,
                        jnp.asarray(_C40), p0, p1, p2, r8)
    acc, _ = _layer_edges(src, dst, sm, h, 1, nck)
    out = _tc_end(acc[0], acc[1], jnp.asarray(_S40), b4.reshape(1, -1))
    return out[:n]


# speedup vs baseline: 48.7988x; 1.0030x over previous
import functools

import jax
import jax.numpy as jnp
import numpy as np
from jax import lax
from jax.experimental import pallas as pl
from jax.experimental.pallas import tpu as pltpu
from jax.experimental.pallas import tpu_sc as plsc

N = 10000
NPAD = 10240
RB = 512
GRID = NPAD // RB
NW = 32
CHUNK = 64
SROWS = NPAD // 16


def _leaky(z):
    return jnp.where(z >= 0, z, 0.2 * z)


_full = lambda s: pl.BlockSpec(s, lambda i: tuple(0 for _ in s))
_row = lambda wd: pl.BlockSpec((RB, wd), lambda i: (i, 0))
_row3 = lambda ld, wd: pl.BlockSpec((ld, RB, wd), lambda i: (0, i, 0))


def _attn_tables(h, as_ref, ad_ref, p0_ref, p1_ref, p2_ref):
    sa = jnp.dot(h, as_ref[...], preferred_element_type=jnp.float32)
    da = jnp.dot(h, ad_ref[...], preferred_element_type=jnp.float32)
    mh = _leaky(sa + da)
    return (jnp.dot(sa, p0_ref[...], preferred_element_type=jnp.float32)
            + jnp.dot(da, p1_ref[...], preferred_element_type=jnp.float32)
            + jnp.dot(mh, p2_ref[...], preferred_element_type=jnp.float32))


def _front_body(p0_ref, p1_ref, p2_ref, x_ref, w_ref, as_ref, ad_ref,
                h_ref, sm_ref):
    h = jnp.dot(x_ref[...], w_ref[...], preferred_element_type=jnp.float32)
    h_ref[...] = h
    sm_ref[...] = _attn_tables(h, as_ref, ad_ref, p0_ref, p1_ref, p2_ref)


def _tc_front(x, w, a_s, a_d, p0, p1, p2):
    outs = [
        jax.ShapeDtypeStruct((NPAD, 128), jnp.float32),
        jax.ShapeDtypeStruct((NPAD, 128), jnp.float32),
    ]
    return pl.pallas_call(
        _front_body,
        grid=(GRID,),
        in_specs=[_full(p0.shape), _full(p1.shape), _full(p2.shape),
                  _row(128), _full(w.shape), _full(a_s.shape), _full(a_d.shape)],
        out_specs=[_row(128), _row(128)],
        out_shape=outs,
    )(p0, p1, p2, x, w, a_s, a_d)


def _mid_body(residual, c40, p0_ref, p1_ref, p2_ref, r8_ref, a0_ref, a1_ref,
              s2_ref, b_ref, xp_ref, w_ref, as_ref, ad_ref, c40_ref,
              x_ref, h_ref, sm_ref):
    ssum = jnp.sum(s2_ref[...], axis=0)
    srep = jnp.dot(ssum, r8_ref[...], preferred_element_type=jnp.float32)
    u = a0_ref[...] + a1_ref[...]
    xi = jnp.maximum(u / (srep + 1e-16) + b_ref[...], 0.0)
    if residual:
        xi = xi + xp_ref[...]
    h = jnp.dot(xi, w_ref[...], preferred_element_type=jnp.float32)
    if c40:
        h = h + c40_ref[...]
    x_ref[...] = xi
    h_ref[...] = h
    sm_ref[...] = _attn_tables(h, as_ref, ad_ref, p0_ref, p1_ref, p2_ref)


def _tc_mid(a0, a1, s2, b, xprev, w, a_s, a_d, residual, c40v,
            p0, p1, p2, r8):
    outs = [
        jax.ShapeDtypeStruct((NPAD, 128), jnp.float32),
        jax.ShapeDtypeStruct((NPAD, 128), jnp.float32),
        jax.ShapeDtypeStruct((NPAD, 128), jnp.float32),
    ]
    return pl.pallas_call(
        functools.partial(_mid_body, residual, c40v is not None),
        grid=(GRID,),
        in_specs=[_full(p0.shape), _full(p1.shape), _full(p2.shape),
                  _full(r8.shape), _row(128), _row(128), _row3(2, 8),
                  _full(b.shape), _row(128),
                  _full(w.shape), _full(a_s.shape), _full(a_d.shape),
                  _full((1, 128))],
        out_specs=[_row(128), _row(128), _row(128)],
        out_shape=outs,
    )(p0, p1, p2, r8, a0, a1, s2, b, xprev, w, a_s, a_d,
      c40v if c40v is not None else jnp.zeros((1, 128), jnp.float32))


def _end_body(s40_ref, a0_ref, a1_ref, b_ref, o_ref):
    t = a0_ref[...] + a1_ref[...]
    srep = jnp.dot(t, s40_ref[...], preferred_element_type=jnp.float32)
    lg = t[:, :40] / (srep + 1e-16) + b_ref[...]
    mx = jnp.max(lg, axis=1, keepdims=True)
    ex = jnp.exp(lg - mx)
    o_ref[...] = lg - mx - jnp.log(jnp.sum(ex, axis=1, keepdims=True))


def _tc_end(a0, a1, s40, b):
    return pl.pallas_call(
        _end_body,
        grid=(GRID,),
        in_specs=[_full(s40.shape), _row(128), _row(128), _full(b.shape)],
        out_specs=_row(40),
        out_shape=jax.ShapeDtypeStruct((NPAD, 40), jnp.float32),
    )(s40, a0, a1, b)


_CP = pltpu.CompilerParams(needs_layout_passes=False)


def _mesh():
    return plsc.VectorSubcoreMesh(core_axis_name="c", subcore_axis_name="s")


def _sc_attn(src, dst, sm_t, heads, n_chunks):
    ew = n_chunks * CHUNK
    epad = NW * ew
    full_den = heads == 8
    assert n_chunks % 2 == 0

    def body(src_hbm, dst_hbm, sm_hbm, *rest):
        if full_den:
            (ee_hbm, sden_hbm, sh_s, src_v0, src_v1, dst_v0, dst_v1,
             rows_s0, rows_s1, rows_d0, rows_d1, ee_buf0, ee_buf1,
             s_part, idx_buf, zbuf,
             si0, si1, sg0, sg1, sw0, sw1) = rest
        else:
            (ee_hbm, src_v0, src_v1, dst_v0, dst_v1,
             rows_s0, rows_s1, rows_d0, rows_d1, ee_buf0, ee_buf1,
             si0, si1, sg0, sg1, sw0, sw1) = rest
        src_v = (src_v0, src_v1)
        dst_v = (dst_v0, dst_v1)
        rows_s = (rows_s0, rows_s1)
        rows_d = (rows_d0, rows_d1)
        ee_buf = (ee_buf0, ee_buf1)
        si = (si0, si1)
        sg = (sg0, sg1)
        sw = (sw0, sw1)
        c = lax.axis_index("c")
        s = lax.axis_index("s")
        wid = s * 2 + c
        zero16 = jnp.zeros((16,), jnp.float32)
        iot = lax.iota(jnp.int32, 16)

        if full_den:
            def zsp(r, _):
                for k in range(8):
                    s_part[r, pl.ds(16 * k, 16)] = zero16
                return 0
            lax.fori_loop(0, 641, zsp, 0)

            def zrow(r, _):
                for k in range(8):
                    zbuf[r, pl.ds(16 * k, 16)] = zero16
                return 0
            lax.fori_loop(0, 8, zrow, 0)

            def zslice(i, _):
                pltpu.sync_copy(zbuf, sh_s.at[pl.ds(s * 40 + i * 8, 8)])
                return 0
            lax.fori_loop(0, 5, zslice, 0)

        def fire_idx(ci, p):
            base = wid * ew + ci * CHUNK
            pltpu.async_copy(src_hbm.at[pl.ds(base, CHUNK)],
                             src_v[p], si[p])
            pltpu.async_copy(dst_hbm.at[pl.ds(base, CHUNK)],
                             dst_v[p], si[p])

        def drain_idx(p):
            pltpu.make_async_copy(src_hbm.at[pl.ds(0, CHUNK)],
                                  src_v[p], si[p]).wait()
            pltpu.make_async_copy(dst_hbm.at[pl.ds(0, CHUNK)],
                                  dst_v[p], si[p]).wait()

        def fire_gather(p):
            pltpu.async_copy(sm_hbm.at[src_v[p]], rows_s[p], sg[p])
            pltpu.async_copy(sm_hbm.at[dst_v[p]], rows_d[p], sg[p])

        def drain_gather(p):
            pltpu.make_async_copy(sm_hbm.at[pl.ds(0, CHUNK)],
                                  rows_s[p], sg[p]).wait()
            pltpu.make_async_copy(sm_hbm.at[pl.ds(0, CHUNK)],
                                  rows_d[p], sg[p]).wait()

        def drain_write(p):
            pltpu.make_async_copy(ee_hbm.at[pl.ds(0, CHUNK * 16)],
                                  ee_buf[p], sw[p]).wait()

        def compute(p, ci):
            @plsc.parallel_loop(0, CHUNK, unroll=4)
            def edge(e):
                z = rows_s[p][e, pl.ds(0, 16)] + rows_d[p][e, pl.ds(16, 16)]
                expo = jnp.exp(_leaky(z) - rows_d[p][e, pl.ds(32, 16)])
                expo = jnp.where(iot < heads, expo, 0.0)
                ee_buf[p][pl.ds(16 * e, 16)] = expo
                if full_den:
                    d_vec = plsc.load_gather(
                        dst_v[p], [jnp.full((16,), e, jnp.int32)])
                    idx = d_vec * 8 + iot
                    plsc.addupdate_scatter(
                        s_part,
                        [lax.shift_right_logical(idx, 7), idx & 127],
                        expo, mask=iot < 8)
            base = wid * ew + ci * CHUNK
            pltpu.async_copy(ee_buf[p],
                             ee_hbm.at[pl.ds(base * 16, CHUNK * 16)], sw[p])

        fire_idx(0, 0)
        drain_idx(0)
        fire_gather(0)
        fire_idx(1, 1)

        def step(i, _):
            ci = 2 * i
            drain_idx(1)
            fire_gather(1)
            drain_gather(0)

            @pl.when(ci >= 2)
            def _():
                drain_write(0)
            compute(0, ci)

            @pl.when(ci + 2 < n_chunks)
            def _():
                fire_idx(ci + 2, 0)

            @pl.when(ci + 2 < n_chunks)
            def _():
                drain_idx(0)
                fire_gather(0)
            drain_gather(1)

            @pl.when(ci >= 2)
            def _():
                drain_write(1)
            compute(1, ci + 1)

            @pl.when(ci + 3 < n_chunks)
            def _():
                fire_idx(ci + 3, 1)
            return 0
        lax.fori_loop(0, n_chunks // 2, step, 0)
        drain_write(0)
        drain_write(1)

        if full_den:
            plsc.subcore_barrier()
            for m in range(5):
                for v in range(8):
                    idx_buf[pl.ds(16 * v, 16)] = iot + (128 * m + 16 * v)
                pltpu.sync_copy(s_part.at[pl.ds(128 * m, 128)],
                                sh_s.at[idx_buf], add=True)
            plsc.subcore_barrier()
            pltpu.sync_copy(sh_s.at[pl.ds(s * 40, 40)],
                            sden_hbm.at[c, pl.ds(s * 40, 40)])

    sems = [pltpu.SemaphoreType.DMA] * 6
    if full_den:
        out_type = [
            jax.ShapeDtypeStruct((epad * 16,), jnp.float32),
            jax.ShapeDtypeStruct((2, SROWS, 128), jnp.float32),
        ]
        scratch = [
            pltpu.VMEM_SHARED((SROWS, 128), jnp.float32),
            pltpu.VMEM((CHUNK,), jnp.int32),
            pltpu.VMEM((CHUNK,), jnp.int32),
            pltpu.VMEM((CHUNK,), jnp.int32),
            pltpu.VMEM((CHUNK,), jnp.int32),
            pltpu.VMEM((CHUNK, 128), jnp.float32),
            pltpu.VMEM((CHUNK, 128), jnp.float32),
            pltpu.VMEM((CHUNK, 128), jnp.float32),
            pltpu.VMEM((CHUNK, 128), jnp.float32),
            pltpu.VMEM((CHUNK * 16,), jnp.float32),
            pltpu.VMEM((CHUNK * 16,), jnp.float32),
            pltpu.VMEM((641, 128), jnp.float32),
            pltpu.VMEM((128,), jnp.int32),
            pltpu.VMEM((8, 128), jnp.float32),
        ] + sems
    else:
        out_type = [jax.ShapeDtypeStruct((epad * 16,), jnp.float32)]
        scratch = [
            pltpu.VMEM((CHUNK,), jnp.int32),
            pltpu.VMEM((CHUNK,), jnp.int32),
            pltpu.VMEM((CHUNK,), jnp.int32),
            pltpu.VMEM((CHUNK,), jnp.int32),
            pltpu.VMEM((CHUNK, 128), jnp.float32),
            pltpu.VMEM((CHUNK, 128), jnp.float32),
            pltpu.VMEM((CHUNK, 128), jnp.float32),
            pltpu.VMEM((CHUNK, 128), jnp.float32),
            pltpu.VMEM((CHUNK * 16,), jnp.float32),
            pltpu.VMEM((CHUNK * 16,), jnp.float32),
        ] + sems
    kern = pl.kernel(body, out_type=out_type, mesh=_mesh(),
                     compiler_params=_CP, scratch_types=scratch)
    return kern(src, dst, sm_t)


def _sc_agg(src, dst, ee, h_t, heads, n_chunks):
    ew = n_chunks * CHUNK
    rows_per_sub = NPAD // 16
    assert n_chunks % 2 == 0

    def body(src_hbm, dst_hbm, ee_hbm, h_hbm, acc_hbm,
             sh_acc, src_v0, src_v1, dst_v0, dst_v1, ee_flat0, ee_flat1,
             rows_h0, rows_h1, out_rows0, out_rows1, zbuf,
             si0, si1, sg0, sg1):
        src_v = (src_v0, src_v1)
        dst_v = (dst_v0, dst_v1)
        ee_flat = (ee_flat0, ee_flat1)
        rows_h = (rows_h0, rows_h1)
        out_rows = (out_rows0, out_rows1)
        si = (si0, si1)
        sg = (sg0, sg1)
        c = lax.axis_index("c")
        s = lax.axis_index("s")
        wid = s * 2 + c
        zero16 = jnp.zeros((16,), jnp.float32)

        def zrow(r, _):
            for k in range(8):
                zbuf[r, pl.ds(16 * k, 16)] = zero16
            return 0
        lax.fori_loop(0, 8, zrow, 0)

        if heads == 1:
            def zo(e, _):
                for p in range(2):
                    for k in range(3, 8):
                        out_rows[p][e, pl.ds(16 * k, 16)] = zero16
                return 0
            lax.fori_loop(0, CHUNK, zo, 0)

        def zslice(i, _):
            pltpu.sync_copy(
                zbuf, sh_acc.at[pl.ds(s * rows_per_sub + i * 8, 8)])
            return 0
        lax.fori_loop(0, rows_per_sub // 8, zslice, 0)
        plsc.subcore_barrier()

        def fire_idx(ci, p):
            base = wid * ew + ci * CHUNK
            pltpu.async_copy(src_hbm.at[pl.ds(base, CHUNK)],
                             src_v[p], si[p])
            pltpu.async_copy(dst_hbm.at[pl.ds(base, CHUNK)],
                             dst_v[p], si[p])
            pltpu.async_copy(ee_hbm.at[pl.ds(base * 16, CHUNK * 16)],
                             ee_flat[p], si[p])

        def drain_idx(p):
            pltpu.make_async_copy(src_hbm.at[pl.ds(0, CHUNK)],
                                  src_v[p], si[p]).wait()
            pltpu.make_async_copy(dst_hbm.at[pl.ds(0, CHUNK)],
                                  dst_v[p], si[p]).wait()
            pltpu.make_async_copy(ee_hbm.at[pl.ds(0, CHUNK * 16)],
                                  ee_flat[p], si[p]).wait()

        def fire_gather(p):
            pltpu.async_copy(h_hbm.at[src_v[p]], rows_h[p], sg[p])

        def drain_gather(p):
            pltpu.make_async_copy(h_hbm.at[pl.ds(0, CHUNK)],
                                  rows_h[p], sg[p]).wait()

        def compute(p, ci):
            @plsc.parallel_loop(0, CHUNK, unroll=4)
            def edge(e):
                nv = 8 if heads == 8 else 3
                for j in range(nv):
                    jj = j if heads == 8 else 0
                    bj = plsc.load_gather(
                        ee_flat[p],
                        [jnp.full((16,), jj, jnp.int32) + 16 * e])
                    out_rows[p][e, pl.ds(16 * j, 16)] = (
                        rows_h[p][e, pl.ds(16 * j, 16)] * bj)
            pltpu.sync_copy(out_rows[p], sh_acc.at[dst_v[p]], add=True)

        fire_idx(0, 0)
        drain_idx(0)
        fire_gather(0)
        fire_idx(1, 1)

        def step(i, _):
            ci = 2 * i
            drain_idx(1)
            fire_gather(1)
            drain_gather(0)
            compute(0, ci)

            @pl.when(ci + 2 < n_chunks)
            def _():
                fire_idx(ci + 2, 0)

            @pl.when(ci + 2 < n_chunks)
            def _():
                drain_idx(0)
                fire_gather(0)
            drain_gather(1)
            compute(1, ci + 1)

            @pl.when(ci + 3 < n_chunks)
            def _():
                fire_idx(ci + 3, 1)
            return 0
        lax.fori_loop(0, n_chunks // 2, step, 0)
        plsc.subcore_barrier()
        pltpu.sync_copy(
            sh_acc.at[pl.ds(s * rows_per_sub, rows_per_sub)],
            acc_hbm.at[c, pl.ds(s * rows_per_sub, rows_per_sub)])

    kern = pl.kernel(
        body,
        out_type=jax.ShapeDtypeStruct((2, NPAD, 128), jnp.float32),
        mesh=_mesh(),
        compiler_params=_CP,
        scratch_types=[
            pltpu.VMEM_SHARED((NPAD, 128), jnp.float32),
            pltpu.VMEM((CHUNK,), jnp.int32),
            pltpu.VMEM((CHUNK,), jnp.int32),
            pltpu.VMEM((CHUNK,), jnp.int32),
            pltpu.VMEM((CHUNK,), jnp.int32),
            pltpu.VMEM((CHUNK * 16,), jnp.float32),
            pltpu.VMEM((CHUNK * 16,), jnp.float32),
            pltpu.VMEM((CHUNK, 128), jnp.float32),
            pltpu.VMEM((CHUNK, 128), jnp.float32),
            pltpu.VMEM((CHUNK, 128), jnp.float32),
            pltpu.VMEM((CHUNK, 128), jnp.float32),
            pltpu.VMEM((8, 128), jnp.float32),
            pltpu.SemaphoreType.DMA,
            pltpu.SemaphoreType.DMA,
            pltpu.SemaphoreType.DMA,
            pltpu.SemaphoreType.DMA,
        ],
    )
    return kern(src, dst, ee, h_t)


def _blockdiag(a):
    bd = (a[:, :, None] * jnp.eye(8, dtype=a.dtype)[:, None, :]).reshape(128, 8)
    return jnp.concatenate([bd, jnp.zeros((128, 8), a.dtype)], axis=1)


def _place(off):
    p = np.zeros((16, 128), np.float32)
    p[np.arange(16), off + np.arange(16)] = 1.0
    return p


_P0, _P1, _P2 = _place(0), _place(16), _place(32)
_R8 = np.kron(np.eye(8, dtype=np.float32), np.ones((1, 16), np.float32))
_S40 = np.zeros((128, 40), np.float32)
_S40[40, :] = 1.0
_C40 = np.zeros((1, 128), np.float32)
_C40[0, 40] = 1.0


def _layer_edges(src, dst, sm, h, heads, nck):
    att = _sc_attn(src, dst, sm, heads, nck)
    if heads == 8:
        ee, sden = att
    else:
        (ee,) = att
        sden = None
    acc = _sc_agg(src, dst, ee, h, heads, nck)
    return acc, sden


def kernel(x, edge_index, W1, as1, ad1, b1, W2, as2, ad2, b2,
           W3, as3, ad3, b3, W4, as4, ad4, b4):
    n = x.shape[0]
    e = edge_index.shape[1]
    etot = e + n
    nck = -(-etot // (NW * CHUNK))
    nck += nck % 2
    epad = NW * CHUNK * nck

    loops = jnp.arange(n, dtype=edge_index.dtype)
    pad = jnp.full((epad - etot,), n, edge_index.dtype)
    src = jnp.concatenate([edge_index[0], loops, pad])
    dst = jnp.concatenate([edge_index[1], loops, pad])

    xp = jnp.pad(x, ((0, NPAD - n), (0, 0)))
    p0, p1, p2 = jnp.asarray(_P0), jnp.asarray(_P1), jnp.asarray(_P2)
    r8 = jnp.asarray(_R8)

    as4p = jnp.zeros((128, 16), jnp.float32).at[:40, 0].set(as4[0])
    ad4p = jnp.zeros((128, 16), jnp.float32).at[:40, 0].set(ad4[0])
    w4p = jnp.pad(W4, ((0, 0), (0, 88)))

    h, sm = _tc_front(xp, W1, _blockdiag(as1), _blockdiag(ad1), p0, p1, p2)
    acc, sden = _layer_edges(src, dst, sm, h, 8, nck)
    x1, h, sm = _tc_mid(acc[0], acc[1], sden.reshape(2, NPAD, 8),
                        b1.reshape(1, -1), xp, W2,
                        _blockdiag(as2), _blockdiag(ad2), False, None,
                        p0, p1, p2, r8)
    acc, sden = _layer_edges(src, dst, sm, h, 8, nck)
    x2, h, sm = _tc_mid(acc[0], acc[1], sden.reshape(2, NPAD, 8),
                        b2.reshape(1, -1), x1, W3,
                        _blockdiag(as3), _blockdiag(ad3), True, None,
                        p0, p1, p2, r8)
    acc, sden = _layer_edges(src, dst, sm, h, 8, nck)
    x3, h, sm = _tc_mid(acc[0], acc[1], sden.reshape(2, NPAD, 8),
                        b3.reshape(1, -1), x2, w4p, as4p, ad4p, True,
                        jnp.asarray(_C40), p0, p1, p2, r8)
    acc, _ = _layer_edges(src, dst, sm, h, 1, nck)
    out = _tc_end(acc[0], acc[1], jnp.asarray(_S40), b4.reshape(1, -1))
    return out[:n]
